# baseline (speedup 1.0000x reference)
import math

import jax
import jax.numpy as jnp
from jax import lax
from jax.experimental import pallas as pl
from jax.experimental.pallas import tpu as pltpu
from jax.experimental.pallas import tpu_sc as plsc

N = 10000
DCOMP = 128
NBR = 32
NLANE = 16
DW = DCOMP // 2
KW = DW // NLANE

NBLK = 32
RPW = 320
NPAD = NBLK * RPW
GR = 4
NG = RPW // GR
SUB = 4
TSUP = NG // SUB
NCORES = 1
XTRA0 = 0
NB1 = NBLK - 16 - XTRA0

LOG2PI = float(math.log(2.0 * math.pi))
P_D = 0.9
C1 = float(-NBR * math.log(1.0 - P_D))
C2 = float(math.log(P_D) - math.log(1.0 - P_D))


def _tc_body(z_ref, s_ref, qpk_ref, p_ref):
    z = z_ref[...]
    s = s_ref[...]
    u = 0.5 * (z * z + s * s) + LOG2PI
    nu = -u
    m = jnp.max(nu, axis=1, keepdims=True)
    e = jnp.exp(nu - m)
    q = e / jnp.sum(e, axis=1, keepdims=True)
    q16 = q.astype(jnp.bfloat16)
    lo = lax.bitcast_convert_type(q16[:, :DW], jnp.uint16).astype(jnp.uint32)
    hi = lax.bitcast_convert_type(q16[:, DW:], jnp.uint16).astype(jnp.uint32)
    qpk_ref[...] = lax.bitcast_convert_type(lo | (hi << 16), jnp.int32)
    p_ref[...] = jnp.sum(q * u, axis=1, keepdims=True)


_TC_BLK = 512
_tc_call = pl.pallas_call(
    _tc_body,
    grid=(NPAD // _TC_BLK,),
    in_specs=[
        pl.BlockSpec((_TC_BLK, DCOMP), lambda i: (i, 0)),
        pl.BlockSpec((_TC_BLK, DCOMP), lambda i: (i, 0)),
    ],
    out_specs=[
        pl.BlockSpec((_TC_BLK, DW), lambda i: (i, 0)),
        pl.BlockSpec((_TC_BLK, 1), lambda i: (i, 0)),
    ],
    out_shape=[
        jax.ShapeDtypeStruct((NPAD, DW), jnp.int32),
        jax.ShapeDtypeStruct((NPAD, 1), jnp.float32),
    ],
)


def _as_f32_lo(w):
    return lax.bitcast_convert_type(w << 16, jnp.float32)


def _as_f32_hi(w):
    return lax.bitcast_convert_type(w, jnp.float32)


def _sc_body(q_hbm, d_hbm, p_hbm, e_hbm,
             qsh, idxg0, idxg1, rows0, rows1, qown, pown, eout,
             sem0, sem1, semi0, semi1):
    cid = lax.axis_index("c")
    sid = lax.axis_index("s")

    @pl.when(sid == 0)
    def _():
        pltpu.sync_copy(q_hbm, qsh)

    plsc.subcore_barrier()

    lanes = lax.broadcasted_iota(jnp.int32, (NLANE,), 0)

    def lane_sum(x):
        dnums = lax.GatherDimensionNumbers(
            offset_dims=(), collapsed_slice_dims=(0,), start_index_map=(0,))
        for d in (1, 2, 4, 8):
            perm = lax.gather(
                x, (lanes ^ d)[:, None], dimension_numbers=dnums,
                slice_sizes=(1,), unique_indices=True,
                mode=lax.GatherScatterMode.PROMISE_IN_BOUNDS)
            x = x + perm
        return x

    def process_block(blk):
        base = blk * RPW
        dbase = blk * NG

        pltpu.sync_copy(q_hbm.at[pl.ds(base, RPW), :], qown)
        pltpu.sync_copy(p_hbm.at[pl.ds(base, RPW)], pown)

        pltpu.sync_copy(d_hbm.at[dbase], idxg0)
        pltpu.async_copy(qsh.at[idxg0], rows0, sem0)
        pltpu.sync_copy(d_hbm.at[dbase + 1], idxg1)
        pltpu.async_copy(qsh.at[idxg1], rows1, sem1)

        def super_body(t, carry):
            cvec = jnp.zeros((NLANE,), jnp.float32)
            for sub in range(SUB):
                g = t * SUB + sub
                if sub % 2 == 0:
                    buf, sem, idxg, semi = rows0, sem0, idxg0, semi0
                else:
                    buf, sem, idxg, semi = rows1, sem1, idxg1, semi1
                pltpu.make_async_copy(qsh.at[idxg], buf, sem).wait()

                @pl.when(g + 2 < NG)
                def _():
                    pltpu.async_copy(d_hbm.at[dbase + g + 2], idxg, semi)

                def jbody(j, acc):
                    new = list(acc)
                    for i in range(GR):
                        r = NBR * i + j
                        for k in range(KW):
                            w = buf[r, pl.ds(NLANE * k, NLANE)]
                            new[2 * (i * KW + k)] = (
                                new[2 * (i * KW + k)] + _as_f32_lo(w))
                            new[2 * (i * KW + k) + 1] = (
                                new[2 * (i * KW + k) + 1] + _as_f32_hi(w))
                    return tuple(new)

                acc = lax.fori_loop(
                    0, NBR, jbody,
                    tuple(jnp.zeros((NLANE,), jnp.float32)
                          for _ in range(2 * GR * KW)))

                lr = t * (SUB * GR) + sub * GR
                for i in range(GR):
                    m = jnp.zeros((NLANE,), jnp.float32)
                    for k in range(KW):
                        qw = qown[lr + i, pl.ds(NLANE * k, NLANE)]
                        m = m + acc[2 * (i * KW + k)] * _as_f32_lo(qw)
                        m = m + acc[2 * (i * KW + k) + 1] * _as_f32_hi(qw)
                    co = lane_sum(m)
                    cvec = jnp.where(lanes == (sub * GR + i), co, cvec)

                @pl.when(g + 2 < NG)
                def _():
                    pltpu.make_async_copy(d_hbm.at[dbase], idxg, semi).wait()
                    pltpu.async_copy(qsh.at[idxg], buf, sem)

            pvec = pown[pl.ds(t * (SUB * GR), SUB * GR)]
            eout[pl.ds(t * (SUB * GR), SUB * GR)] = pvec + C1 - C2 * cvec
            return carry

        lax.fori_loop(0, TSUP, super_body, 0)
        pltpu.sync_copy(eout, e_hbm.at[pl.ds(base, RPW)])

    if NCORES == 1:
        process_block(sid)
        process_block(16 + sid)
    else:
        blk1 = jnp.where(cid == 1, sid, 16 + XTRA0 + sid)
        do1 = jnp.logical_or(cid == 1, sid < NB1)
        blk2 = 16 + sid
        do2 = jnp.logical_and(cid == 1, sid < XTRA0)

        @pl.when(do1)
        def _():
            process_block(blk1)

        @pl.when(do2)
        def _():
            process_block(blk2)


def _make_sc_call():
    return pl.kernel(
        _sc_body,
        out_type=jax.ShapeDtypeStruct((NPAD,), jnp.float32),
        mesh=plsc.VectorSubcoreMesh(core_axis_name="c", subcore_axis_name="s",
                                    num_cores=NCORES),
        compiler_params=pltpu.CompilerParams(use_tc_tiling_on_sc=False),
        scratch_types=[
            pltpu.VMEM_SHARED((NPAD, DW), jnp.int32),
            pltpu.VMEM((128,), jnp.int32),
            pltpu.VMEM((128,), jnp.int32),
            pltpu.VMEM((GR * NBR, DW), jnp.int32),
            pltpu.VMEM((GR * NBR, DW), jnp.int32),
            pltpu.VMEM((RPW, DW), jnp.int32),
            pltpu.VMEM((RPW,), jnp.float32),
            pltpu.VMEM((RPW,), jnp.float32),
            pltpu.SemaphoreType.DMA,
            pltpu.SemaphoreType.DMA,
            pltpu.SemaphoreType.DMA,
            pltpu.SemaphoreType.DMA,
        ],
    )


@jax.jit
def kernel(Z, S, D):
    qpk, p = _tc_call(Z, S)
    d2 = jnp.pad(D, ((0, NPAD - N), (0, 0))).reshape(NPAD * NBR // 128, 128)
    e = _make_sc_call()(qpk, d2, p.reshape(NPAD))
    return e[:N]


# speedup vs baseline: 1.4033x; 1.4033x over previous
import math

import jax
import jax.numpy as jnp
from jax import lax
from jax.experimental import pallas as pl
from jax.experimental.pallas import tpu as pltpu
from jax.experimental.pallas import tpu_sc as plsc

N = 10000
DCOMP = 128
NBR = 32
NLANE = 16
DW = DCOMP // 2
KW = DW // NLANE

NBLK = 32
RPW = 320
NPAD = NBLK * RPW
GR = 4
NG = RPW // GR
SUB = 4
TSUP = NG // SUB
NCORES = 2
XTRA0 = 0
NB1 = NBLK - 16 - XTRA0

LOG2PI = float(math.log(2.0 * math.pi))
P_D = 0.9
C1 = float(-NBR * math.log(1.0 - P_D))
C2 = float(math.log(P_D) - math.log(1.0 - P_D))


def _tc_body(z_ref, s_ref, qpk_ref, p_ref):
    z = z_ref[...]
    s = s_ref[...]
    u = 0.5 * (z * z + s * s) + LOG2PI
    nu = -u
    m = jnp.max(nu, axis=1, keepdims=True)
    e = jnp.exp(nu - m)
    q = e / jnp.sum(e, axis=1, keepdims=True)
    q16 = q.astype(jnp.bfloat16)
    lo = lax.bitcast_convert_type(q16[:, :DW], jnp.uint16).astype(jnp.uint32)
    hi = lax.bitcast_convert_type(q16[:, DW:], jnp.uint16).astype(jnp.uint32)
    qpk_ref[...] = lax.bitcast_convert_type(lo | (hi << 16), jnp.int32)
    p_ref[...] = jnp.sum(q * u, axis=1, keepdims=True)


_TC_BLK = 512
_tc_call = pl.pallas_call(
    _tc_body,
    grid=(NPAD // _TC_BLK,),
    in_specs=[
        pl.BlockSpec((_TC_BLK, DCOMP), lambda i: (i, 0)),
        pl.BlockSpec((_TC_BLK, DCOMP), lambda i: (i, 0)),
    ],
    out_specs=[
        pl.BlockSpec((_TC_BLK, DW), lambda i: (i, 0)),
        pl.BlockSpec((_TC_BLK, 1), lambda i: (i, 0)),
    ],
    out_shape=[
        jax.ShapeDtypeStruct((NPAD, DW), jnp.int32),
        jax.ShapeDtypeStruct((NPAD, 1), jnp.float32),
    ],
)


def _as_f32_lo(w):
    return lax.bitcast_convert_type(w << 16, jnp.float32)


def _as_f32_hi(w):
    return lax.bitcast_convert_type(w, jnp.float32)


def _sc_body(q_hbm, d_hbm, p_hbm, e_hbm,
             qsh, idxg0, idxg1, rows0, rows1, qown, pown, eout,
             sem0, sem1, semi0, semi1):
    cid = lax.axis_index("c")
    sid = lax.axis_index("s")

    @pl.when(sid == 0)
    def _():
        pltpu.sync_copy(q_hbm, qsh)

    plsc.subcore_barrier()

    lanes = lax.broadcasted_iota(jnp.int32, (NLANE,), 0)

    def lane_sum(x):
        dnums = lax.GatherDimensionNumbers(
            offset_dims=(), collapsed_slice_dims=(0,), start_index_map=(0,))
        for d in (1, 2, 4, 8):
            perm = lax.gather(
                x, (lanes ^ d)[:, None], dimension_numbers=dnums,
                slice_sizes=(1,), unique_indices=True,
                mode=lax.GatherScatterMode.PROMISE_IN_BOUNDS)
            x = x + perm
        return x

    def process_block(blk):
        base = blk * RPW
        dbase = blk * NG

        pltpu.sync_copy(q_hbm.at[pl.ds(base, RPW), :], qown)
        pltpu.sync_copy(p_hbm.at[pl.ds(base, RPW)], pown)

        pltpu.sync_copy(d_hbm.at[dbase], idxg0)
        pltpu.async_copy(qsh.at[idxg0], rows0, sem0)
        pltpu.sync_copy(d_hbm.at[dbase + 1], idxg1)
        pltpu.async_copy(qsh.at[idxg1], rows1, sem1)

        def super_body(t, carry):
            cvec = jnp.zeros((NLANE,), jnp.float32)
            for sub in range(SUB):
                g = t * SUB + sub
                if sub % 2 == 0:
                    buf, sem, idxg, semi = rows0, sem0, idxg0, semi0
                else:
                    buf, sem, idxg, semi = rows1, sem1, idxg1, semi1
                pltpu.make_async_copy(qsh.at[idxg], buf, sem).wait()

                @pl.when(g + 2 < NG)
                def _():
                    pltpu.async_copy(d_hbm.at[dbase + g + 2], idxg, semi)

                def jbody(j, acc):
                    new = list(acc)
                    for i in range(GR):
                        r = NBR * i + j
                        for k in range(KW):
                            w = buf[r, pl.ds(NLANE * k, NLANE)]
                            new[2 * (i * KW + k)] = (
                                new[2 * (i * KW + k)] + _as_f32_lo(w))
                            new[2 * (i * KW + k) + 1] = (
                                new[2 * (i * KW + k) + 1] + _as_f32_hi(w))
                    return tuple(new)

                acc = lax.fori_loop(
                    0, NBR, jbody,
                    tuple(jnp.zeros((NLANE,), jnp.float32)
                          for _ in range(2 * GR * KW)))

                lr = t * (SUB * GR) + sub * GR
                for i in range(GR):
                    m = jnp.zeros((NLANE,), jnp.float32)
                    for k in range(KW):
                        qw = qown[lr + i, pl.ds(NLANE * k, NLANE)]
                        m = m + acc[2 * (i * KW + k)] * _as_f32_lo(qw)
                        m = m + acc[2 * (i * KW + k) + 1] * _as_f32_hi(qw)
                    co = lane_sum(m)
                    cvec = jnp.where(lanes == (sub * GR + i), co, cvec)

                @pl.when(g + 2 < NG)
                def _():
                    pltpu.make_async_copy(d_hbm.at[dbase], idxg, semi).wait()
                    pltpu.async_copy(qsh.at[idxg], buf, sem)

            pvec = pown[pl.ds(t * (SUB * GR), SUB * GR)]
            eout[pl.ds(t * (SUB * GR), SUB * GR)] = pvec + C1 - C2 * cvec
            return carry

        lax.fori_loop(0, TSUP, super_body, 0)
        pltpu.sync_copy(eout, e_hbm.at[pl.ds(base, RPW)])

    if NCORES == 1:
        process_block(sid)
        process_block(16 + sid)
    else:
        blk1 = jnp.where(cid == 1, sid, 16 + XTRA0 + sid)
        do1 = jnp.logical_or(cid == 1, sid < NB1)
        blk2 = 16 + sid
        do2 = jnp.logical_and(cid == 1, sid < XTRA0)

        @pl.when(do1)
        def _():
            process_block(blk1)

        @pl.when(do2)
        def _():
            process_block(blk2)


def _make_sc_call():
    return pl.kernel(
        _sc_body,
        out_type=jax.ShapeDtypeStruct((NPAD,), jnp.float32),
        mesh=plsc.VectorSubcoreMesh(core_axis_name="c", subcore_axis_name="s",
                                    num_cores=NCORES),
        compiler_params=pltpu.CompilerParams(use_tc_tiling_on_sc=False),
        scratch_types=[
            pltpu.VMEM_SHARED((NPAD, DW), jnp.int32),
            pltpu.VMEM((128,), jnp.int32),
            pltpu.VMEM((128,), jnp.int32),
            pltpu.VMEM((GR * NBR, DW), jnp.int32),
            pltpu.VMEM((GR * NBR, DW), jnp.int32),
            pltpu.VMEM((RPW, DW), jnp.int32),
            pltpu.VMEM((RPW,), jnp.float32),
            pltpu.VMEM((RPW,), jnp.float32),
            pltpu.SemaphoreType.DMA,
            pltpu.SemaphoreType.DMA,
            pltpu.SemaphoreType.DMA,
            pltpu.SemaphoreType.DMA,
        ],
    )


@jax.jit
def kernel(Z, S, D):
    qpk, p = _tc_call(Z, S)
    d2 = jnp.pad(D, ((0, NPAD - N), (0, 0))).reshape(NPAD * NBR // 128, 128)
    e = _make_sc_call()(qpk, d2, p.reshape(NPAD))
    return e[:N]


# speedup vs baseline: 1.4164x; 1.0094x over previous
import math

import jax
import jax.numpy as jnp
from jax import lax
from jax.experimental import pallas as pl
from jax.experimental.pallas import tpu as pltpu
from jax.experimental.pallas import tpu_sc as plsc

N = 10000
DCOMP = 128
NBR = 32
NLANE = 16
DW = DCOMP // 4
KW = DW // NLANE
QSCALE = 255.0
INVQ2 = float(1.0 / (255.0 * 255.0))

NBLK = 32
RPW = 320
NPAD = NBLK * RPW
GR = 4
NG = RPW // GR
SUB = 4
TSUP = NG // SUB
NCORES = 2
XTRA0 = 0
NB1 = NBLK - 16 - XTRA0

LOG2PI = float(math.log(2.0 * math.pi))
P_D = 0.9
C1 = float(-NBR * math.log(1.0 - P_D))
C2 = float(math.log(P_D) - math.log(1.0 - P_D))


def _tc_body(z_ref, s_ref, qpk_ref, p_ref):
    z = z_ref[...]
    s = s_ref[...]
    u = 0.5 * (z * z + s * s) + LOG2PI
    nu = -u
    m = jnp.max(nu, axis=1, keepdims=True)
    e = jnp.exp(nu - m)
    q = e / jnp.sum(e, axis=1, keepdims=True)
    q8 = jnp.clip(jnp.round(q * QSCALE), 0.0, 255.0).astype(jnp.uint32)
    w = (q8[:, 0 * DW:1 * DW]
         | (q8[:, 1 * DW:2 * DW] << 8)
         | (q8[:, 2 * DW:3 * DW] << 16)
         | (q8[:, 3 * DW:4 * DW] << 24))
    qpk_ref[...] = lax.bitcast_convert_type(w, jnp.int32)
    p_ref[...] = jnp.sum(q * u, axis=1, keepdims=True)


_TC_BLK = 512
_tc_call = pl.pallas_call(
    _tc_body,
    grid=(NPAD // _TC_BLK,),
    in_specs=[
        pl.BlockSpec((_TC_BLK, DCOMP), lambda i: (i, 0)),
        pl.BlockSpec((_TC_BLK, DCOMP), lambda i: (i, 0)),
    ],
    out_specs=[
        pl.BlockSpec((_TC_BLK, DW), lambda i: (i, 0)),
        pl.BlockSpec((_TC_BLK, 1), lambda i: (i, 0)),
    ],
    out_shape=[
        jax.ShapeDtypeStruct((NPAD, DW), jnp.int32),
        jax.ShapeDtypeStruct((NPAD, 1), jnp.float32),
    ],
)


_MSK16 = 0x00FF00FF


def _sc_body(q_hbm, d_hbm, p_hbm, e_hbm,
             qsh, idxg0, idxg1, rows0, rows1, qown, pown, eout,
             sem0, sem1, semi0, semi1):
    cid = lax.axis_index("c")
    sid = lax.axis_index("s")

    @pl.when(sid == 0)
    def _():
        pltpu.sync_copy(q_hbm, qsh)

    plsc.subcore_barrier()

    lanes = lax.broadcasted_iota(jnp.int32, (NLANE,), 0)

    def lane_sum(x):
        dnums = lax.GatherDimensionNumbers(
            offset_dims=(), collapsed_slice_dims=(0,), start_index_map=(0,))
        for d in (1, 2, 4, 8):
            perm = lax.gather(
                x, (lanes ^ d)[:, None], dimension_numbers=dnums,
                slice_sizes=(1,), unique_indices=True,
                mode=lax.GatherScatterMode.PROMISE_IN_BOUNDS)
            x = x + perm
        return x

    def process_block(blk):
        base = blk * RPW
        dbase = blk * NG

        pltpu.sync_copy(q_hbm.at[pl.ds(base, RPW), :], qown)
        pltpu.sync_copy(p_hbm.at[pl.ds(base, RPW)], pown)

        pltpu.sync_copy(d_hbm.at[dbase], idxg0)
        pltpu.async_copy(qsh.at[idxg0], rows0, sem0)
        pltpu.sync_copy(d_hbm.at[dbase + 1], idxg1)
        pltpu.async_copy(qsh.at[idxg1], rows1, sem1)

        def super_body(t, carry):
            cvec = jnp.zeros((NLANE,), jnp.float32)
            for sub in range(SUB):
                g = t * SUB + sub
                if sub % 2 == 0:
                    buf, sem, idxg, semi = rows0, sem0, idxg0, semi0
                else:
                    buf, sem, idxg, semi = rows1, sem1, idxg1, semi1
                pltpu.make_async_copy(qsh.at[idxg], buf, sem).wait()

                @pl.when(g + 2 < NG)
                def _():
                    pltpu.async_copy(d_hbm.at[dbase + g + 2], idxg, semi)

                def jbody(j, acc):
                    new = list(acc)
                    for i in range(GR):
                        r = NBR * i + j
                        for k in range(KW):
                            w = buf[r, pl.ds(NLANE * k, NLANE)]
                            new[2 * (i * KW + k)] = (
                                new[2 * (i * KW + k)] + (w & _MSK16))
                            new[2 * (i * KW + k) + 1] = (
                                new[2 * (i * KW + k) + 1]
                                + ((w >> 8) & _MSK16))
                    return tuple(new)

                acc = lax.fori_loop(
                    0, NBR, jbody,
                    tuple(jnp.zeros((NLANE,), jnp.int32)
                          for _ in range(2 * GR * KW)))

                lr = t * (SUB * GR) + sub * GR
                for i in range(GR):
                    m = jnp.zeros((NLANE,), jnp.int32)
                    for k in range(KW):
                        a02 = acc[2 * (i * KW + k)]
                        a13 = acc[2 * (i * KW + k) + 1]
                        qw = qown[lr + i, pl.ds(NLANE * k, NLANE)]
                        m = m + (a02 & 0xFFFF) * (qw & 0xFF)
                        m = m + (a13 & 0xFFFF) * ((qw >> 8) & 0xFF)
                        m = m + (a02 >> 16) * ((qw >> 16) & 0xFF)
                        m = m + (a13 >> 16) * ((qw >> 24) & 0xFF)
                    co = lane_sum(m).astype(jnp.float32) * INVQ2
                    cvec = jnp.where(lanes == (sub * GR + i), co, cvec)

                @pl.when(g + 2 < NG)
                def _():
                    pltpu.make_async_copy(d_hbm.at[dbase], idxg, semi).wait()
                    pltpu.async_copy(qsh.at[idxg], buf, sem)

            pvec = pown[pl.ds(t * (SUB * GR), SUB * GR)]
            eout[pl.ds(t * (SUB * GR), SUB * GR)] = pvec + C1 - C2 * cvec
            return carry

        lax.fori_loop(0, TSUP, super_body, 0)
        pltpu.sync_copy(eout, e_hbm.at[pl.ds(base, RPW)])

    if NCORES == 1:
        process_block(sid)
        process_block(16 + sid)
    else:
        blk1 = jnp.where(cid == 1, sid, 16 + XTRA0 + sid)
        do1 = jnp.logical_or(cid == 1, sid < NB1)
        blk2 = 16 + sid
        do2 = jnp.logical_and(cid == 1, sid < XTRA0)

        @pl.when(do1)
        def _():
            process_block(blk1)

        @pl.when(do2)
        def _():
            process_block(blk2)


def _make_sc_call():
    return pl.kernel(
        _sc_body,
        out_type=jax.ShapeDtypeStruct((NPAD,), jnp.float32),
        mesh=plsc.VectorSubcoreMesh(core_axis_name="c", subcore_axis_name="s",
                                    num_cores=NCORES),
        compiler_params=pltpu.CompilerParams(use_tc_tiling_on_sc=False),
        scratch_types=[
            pltpu.VMEM_SHARED((NPAD, DW), jnp.int32),
            pltpu.VMEM((128,), jnp.int32),
            pltpu.VMEM((128,), jnp.int32),
            pltpu.VMEM((GR * NBR, DW), jnp.int32),
            pltpu.VMEM((GR * NBR, DW), jnp.int32),
            pltpu.VMEM((RPW, DW), jnp.int32),
            pltpu.VMEM((RPW,), jnp.float32),
            pltpu.VMEM((RPW,), jnp.float32),
            pltpu.SemaphoreType.DMA,
            pltpu.SemaphoreType.DMA,
            pltpu.SemaphoreType.DMA,
            pltpu.SemaphoreType.DMA,
        ],
    )


@jax.jit
def kernel(Z, S, D):
    qpk, p = _tc_call(Z, S)
    d2 = jnp.pad(D, ((0, NPAD - N), (0, 0))).reshape(NPAD * NBR // 128, 128)
    e = _make_sc_call()(qpk, d2, p.reshape(NPAD))
    return e[:N]
